# Initial kernel scaffold; baseline (speedup 1.0000x reference)
import jax
import jax.numpy as jnp
from jax.experimental import pallas as pl


def kernel(hidden, mode_ids, router_w, router_b, shared_w1, shared_b1, shared_w2, shared_b2, routed_w1, routed_b1, routed_w2, routed_b2, mode_w1, mode_b1, mode_w2, mode_b2):
    raise NotImplementedError("write your pallas kernel here")


# speedup vs baseline: 2.3271x; 2.3271x over previous
import functools

import jax
import jax.numpy as jnp
from jax.experimental import pallas as pl
from jax.experimental.pallas import tpu as pltpu

B, S, D, H, E, K, M = 1, 2048, 768, 1536, 8, 2, 4
NEXP = 1 + M + E
TOK_TILE = 1024
NTILES = S // TOK_TILE


def _dense_moe_kernel(x_ref, mode_ref, rw_ref, rb_ref, w1_ref, b1_ref, w2_ref,
                      b2_ref, out_ref, logits_ref, tidx_ref, tprob_ref,
                      acc_ref, scale_ref):
    e = pl.program_id(0)
    i = pl.program_id(1)

    @pl.when(e == 0)
    def _router():
        x = x_ref[...]
        logits = jax.lax.dot_general(
            x, rw_ref[...], (((1,), (1,)), ((), ())),
            preferred_element_type=jnp.float32) + rb_ref[...][None, :]
        logits_ref[...] = logits
        mx = jnp.max(logits, axis=1, keepdims=True)
        ex = jnp.exp(logits - mx)
        probs = ex / jnp.sum(ex, axis=1, keepdims=True)
        iota = jax.lax.broadcasted_iota(jnp.int32, (TOK_TILE, E), 1)
        p0 = jnp.max(probs, axis=1, keepdims=True)
        i0 = jnp.min(jnp.where(probs == p0, iota, E), axis=1, keepdims=True)
        masked = jnp.where(iota == i0, -jnp.inf, probs)
        p1 = jnp.max(masked, axis=1, keepdims=True)
        i1 = jnp.min(jnp.where(masked == p1, iota, E), axis=1, keepdims=True)
        tidx_ref[...] = jnp.concatenate([i0, i1], axis=1)
        tprob_ref[...] = jnp.concatenate([p0, p1], axis=1)
        mode = mode_ref[...]
        miota = jax.lax.broadcasted_iota(jnp.int32, (TOK_TILE, M), 1)
        mmask = (mode == miota).astype(jnp.float32)
        gates = probs * ((iota == i0) | (iota == i1)).astype(jnp.float32)
        scale_ref[pl.ds(i * TOK_TILE, TOK_TILE), :] = jnp.concatenate(
            [jnp.ones((TOK_TILE, 1), jnp.float32), mmask, gates], axis=1)

    x = x_ref[...]
    h = jax.lax.dot_general(x, w1_ref[0], (((1,), (1,)), ((), ())),
                            preferred_element_type=jnp.float32)
    h = h + b1_ref[0]
    h = 0.5 * h * (1.0 + jax.lax.erf(h * 0.7071067811865476))
    y = jax.lax.dot_general(h, w2_ref[0], (((1,), (1,)), ((), ())),
                            preferred_element_type=jnp.float32)
    y = y + b2_ref[0]
    sc_tile = scale_ref[pl.ds(i * TOK_TILE, TOK_TILE), :]
    lane = jax.lax.broadcasted_iota(jnp.int32, (TOK_TILE, NEXP), 1)
    s = jnp.sum(jnp.where(lane == e, sc_tile, 0.0), axis=1, keepdims=True)
    contrib = y * s

    @pl.when(e == 0)
    def _init():
        acc_ref[pl.ds(i * TOK_TILE, TOK_TILE), :] = contrib

    @pl.when(e > 0)
    def _accum():
        acc_ref[pl.ds(i * TOK_TILE, TOK_TILE), :] += contrib

    @pl.when(e == NEXP - 1)
    def _emit():
        out_ref[...] = acc_ref[pl.ds(i * TOK_TILE, TOK_TILE), :]


def kernel(hidden, mode_ids, router_w, router_b, shared_w1, shared_b1,
           shared_w2, shared_b2, routed_w1, routed_b1, routed_w2, routed_b2,
           mode_w1, mode_b1, mode_w2, mode_b2):
    flat = hidden.reshape(S, D)
    mode_flat = mode_ids.reshape(S, 1).astype(jnp.int32)
    big_w1 = jnp.concatenate([shared_w1, mode_w1, routed_w1], axis=0)
    big_b1 = jnp.concatenate([shared_b1, mode_b1, routed_b1],
                             axis=0).reshape(NEXP, 1, H)
    big_w2 = jnp.concatenate([shared_w2, mode_w2, routed_w2], axis=0)
    big_b2 = jnp.concatenate([shared_b2, mode_b2, routed_b2],
                             axis=0).reshape(NEXP, 1, D)

    grid = (NEXP, NTILES)
    out, logits, tidx, tprob = pl.pallas_call(
        _dense_moe_kernel,
        grid=grid,
        in_specs=[
            pl.BlockSpec((TOK_TILE, D), lambda e, i: (i, 0)),
            pl.BlockSpec((TOK_TILE, 1), lambda e, i: (i, 0)),
            pl.BlockSpec((E, D), lambda e, i: (0, 0)),
            pl.BlockSpec((E,), lambda e, i: (0,)),
            pl.BlockSpec((1, H, D), lambda e, i: (e, 0, 0)),
            pl.BlockSpec((1, 1, H), lambda e, i: (e, 0, 0)),
            pl.BlockSpec((1, D, H), lambda e, i: (e, 0, 0)),
            pl.BlockSpec((1, 1, D), lambda e, i: (e, 0, 0)),
        ],
        out_specs=[
            pl.BlockSpec((TOK_TILE, D), lambda e, i: (i, 0)),
            pl.BlockSpec((TOK_TILE, E), lambda e, i: (i, 0)),
            pl.BlockSpec((TOK_TILE, K), lambda e, i: (i, 0)),
            pl.BlockSpec((TOK_TILE, K), lambda e, i: (i, 0)),
        ],
        out_shape=[
            jax.ShapeDtypeStruct((S, D), jnp.float32),
            jax.ShapeDtypeStruct((S, E), jnp.float32),
            jax.ShapeDtypeStruct((S, K), jnp.int32),
            jax.ShapeDtypeStruct((S, K), jnp.float32),
        ],
        scratch_shapes=[
            pltpu.VMEM((S, D), jnp.float32),
            pltpu.VMEM((S, NEXP), jnp.float32),
        ],
    )(flat, mode_flat, router_w, router_b, big_w1, big_b1, big_w2, big_b2)

    return (out.reshape(B, S, D), logits.reshape(B, S, E),
            tidx.reshape(B, S, K), tprob.reshape(B, S, K))
